# baseline (speedup 1.0000x reference)
import functools

import jax
import jax.numpy as jnp
from jax.experimental import pallas as pl
from jax.experimental.pallas import tpu as pltpu

_NUM_BINS = 15
_LANES = 128


def _lane_fold(a):
    t = a.shape[1]
    while t > _LANES:
        half = t // 2
        a = a[:, :half] + a[:, half:]
        t = half
    return a


def _ece_body(nb, c_dim, n_total, bnd_ref, sm_ref, lab_ref, out_ref,
              cnt_ref, acc_ref, cfs_ref):
    j = pl.program_id(0)

    @pl.when(j == 0)
    def _init():
        zeros = jnp.zeros(cnt_ref.shape, jnp.float32)
        cnt_ref[...] = zeros
        acc_ref[...] = zeros
        cfs_ref[...] = zeros

    t = sm_ref.shape[2]
    chunk = 2048 if t % 2048 == 0 else t
    for q in range(t // chunk):
        sl = pl.ds(q * chunk, chunk)
        bq = sm_ref[0, :, sl]
        sbest = jnp.full(bq.shape, float(c_dim - 1), jnp.float32)
        for c in range(1, c_dim):
            xc = sm_ref[c, :, sl]
            gt = xc > bq
            bq = jnp.maximum(xc, bq)
            sbest = jnp.where(gt, float(c_dim - 1 - c), sbest)

        target = float(c_dim - 1) - lab_ref[:, sl].astype(jnp.float32)
        cq = (sbest == target).astype(jnp.float32)

        for i in range(_NUM_BINS + 1):
            m = bq > bnd_ref[i]
            cnt_ref[i] += _lane_fold(m.astype(jnp.float32))
            acc_ref[i] += _lane_fold(jnp.where(m, cq, 0.0))
            cfs_ref[i] += _lane_fold(jnp.where(m, bq, 0.0))

    @pl.when(j == nb - 1)
    def _fin():
        cnt = jnp.sum(cnt_ref[...], axis=2)
        accs = jnp.sum(acc_ref[...], axis=2)
        cfss = jnp.sum(cfs_ref[...], axis=2)
        count = cnt[:-1] - cnt[1:]
        prop = count / float(n_total)
        denom = jnp.maximum(count, 1.0)
        acc_b = (accs[:-1] - accs[1:]) / denom
        cfs_b = (cfss[:-1] - cfss[1:]) / denom
        contrib = jnp.where(count > 0.0,
                            jnp.abs(cfs_b - acc_b) * prop, 0.0)
        ece = jnp.sum(contrib, axis=0)
        out_ref[...] = jnp.broadcast_to(ece[:, None], out_ref.shape)


def kernel(edl_u, softmax, label):
    del edl_u
    b_dim, c_dim, n = softmax.shape
    sm_t = jnp.transpose(softmax, (1, 0, 2))
    tile = 8192
    while n % tile:
        tile //= 2
    nb = n // tile

    label = label.astype(jnp.int32)
    bnd = jnp.linspace(0.0, 1.0, _NUM_BINS + 1, dtype=jnp.float32)

    body = functools.partial(_ece_body, nb, c_dim, n)
    out = pl.pallas_call(
        body,
        grid=(nb,),
        in_specs=[
            pl.BlockSpec(memory_space=pltpu.SMEM),
            pl.BlockSpec((c_dim, b_dim, tile), lambda j: (0, 0, j)),
            pl.BlockSpec((b_dim, tile), lambda j: (0, j)),
        ],
        out_specs=pl.BlockSpec((b_dim, _LANES), lambda j: (0, 0)),
        out_shape=jax.ShapeDtypeStruct((b_dim, _LANES), jnp.float32),
        scratch_shapes=[pltpu.VMEM((_NUM_BINS + 1, b_dim, _LANES),
                                   jnp.float32) for _ in range(3)],
    )(bnd, sm_t, label)
    return out[:, 0]
